# Initial kernel scaffold; baseline (speedup 1.0000x reference)
import jax
import jax.numpy as jnp
from jax.experimental import pallas as pl

_B, _E, _C, _K = 8, 576, 256, 8192
_BETA = 0.25


def kernel(x, y, out_codebook, quantization_noise_std):
    latents = x
    flat = latents.reshape(_B * _E, _C)
    dist = (
        jnp.sum(flat ** 2, axis=1, keepdims=True)
        + jnp.sum(out_codebook ** 2, axis=1)
        - 2.0 * (flat @ out_codebook.T)
    )
    inds = jnp.argmin(dist, axis=1, keepdims=True)
    noise = jnp.round(
        jax.random.normal(jax.random.key(1), inds.shape, dtype=jnp.float32)
        * quantization_noise_std
    ).astype(inds.dtype)
    inds_noisy = jnp.clip(inds + noise, 0, _K - 1)
    quantized = []
    for i in (inds, inds_noisy):
        oh = jax.nn.one_hot(i[:, 0], _K, dtype=jnp.float32)
        quantized.append((oh @ out_codebook).reshape(_B, _E, _C))
    q_det, q_noisy = quantized
    out = latents + jax.lax.stop_gradient(q_noisy - latents)
    recon = jnp.mean((out - y) ** 2)
    commit = _BETA * jnp.mean((latents - jax.lax.stop_gradient(q_det)) ** 2)
    codebook_loss = jnp.mean((q_noisy - jax.lax.stop_gradient(latents)) ** 2)
    return out, recon + commit + codebook_loss


# speedup vs baseline: 1.2207x; 1.2207x over previous
import functools

import jax
import jax.numpy as jnp
from jax.experimental import pallas as pl
from jax.experimental.pallas import tpu as pltpu
from jax.experimental.pallas import tpu_sc as plsc

_B, _E, _C, _K = 8, 576, 256, 8192
_BETA = 0.25
_N = _B * _E
_R = 512
_KT = 1024

_NC, _NS = 2, 16
_NW = _NC * _NS
_BPW = _N // _NW
_CH = 72


def _argmin_body(x_ref, w_ref, o_ref, bv_ref, bi_ref):
    k = pl.program_id(1)
    xb = x_ref[...]
    wb = w_ref[...]
    a = jnp.sum(xb * xb, axis=1, keepdims=True)
    b = jnp.sum(wb * wb, axis=1)
    m = jax.lax.dot_general(xb, wb, (((1,), (1,)), ((), ())))
    dist = (a + b[None, :]) - 2.0 * m
    lmin = jnp.min(dist, axis=1, keepdims=True)
    iota = jax.lax.broadcasted_iota(jnp.int32, (_R, _KT), 1)
    lidx = jnp.min(jnp.where(dist == lmin, iota, _K), axis=1, keepdims=True) + k * _KT

    @pl.when(k == 0)
    def _():
        bv_ref[...] = lmin
        bi_ref[...] = lidx

    @pl.when(k > 0)
    def _():
        better = lmin < bv_ref[...]
        bv_ref[...] = jnp.where(better, lmin, bv_ref[...])
        bi_ref[...] = jnp.where(better, lidx, bi_ref[...])

    @pl.when(k == _K // _KT - 1)
    def _():
        o_ref[...] = bi_ref[...]


def _nearest_inds(flat, codebook):
    return pl.pallas_call(
        _argmin_body,
        grid=(_N // _R, _K // _KT),
        in_specs=[
            pl.BlockSpec((_R, _C), lambda r, k: (r, 0)),
            pl.BlockSpec((_KT, _C), lambda r, k: (k, 0)),
        ],
        out_specs=pl.BlockSpec((_R, 1), lambda r, k: (r, 0)),
        out_shape=jax.ShapeDtypeStruct((_N, 1), jnp.int32),
        scratch_shapes=[
            pltpu.VMEM((_R, 1), jnp.float32),
            pltpu.VMEM((_R, 1), jnp.int32),
        ],
        compiler_params=pltpu.CompilerParams(
            dimension_semantics=("arbitrary", "arbitrary")
        ),
    )(flat, codebook)


@functools.partial(
    pl.kernel,
    mesh=plsc.VectorSubcoreMesh(core_axis_name="c", subcore_axis_name="s"),
    out_type=jax.ShapeDtypeStruct((_N, _C), jnp.float32),
    scratch_types=[
        pltpu.VMEM((_BPW,), jnp.int32),
        pltpu.VMEM((_CH, _C), jnp.float32),
        pltpu.SemaphoreType.DMA,
    ],
)
def _gather_sc(table_hbm, idx_hbm, out_hbm, idx_v, rows_v, sem):
    wid = jax.lax.axis_index("s") * _NC + jax.lax.axis_index("c")
    base = wid * _BPW
    pltpu.sync_copy(idx_hbm.at[pl.ds(base, _BPW)], idx_v)
    for j in range(_BPW // _CH):
        pltpu.async_copy(table_hbm.at[idx_v.at[pl.ds(j * _CH, _CH)]], rows_v, sem).wait()
        pltpu.sync_copy(rows_v, out_hbm.at[pl.ds(base + j * _CH, _CH)])


def _loss_body(x_ref, y_ref, q_ref, out_ref, loss_ref, acc_ref):
    i = pl.program_id(0)
    xv = x_ref[...]
    yv = y_ref[...]
    qv = q_ref[...]
    outv = xv + (qv - xv)
    out_ref[...] = outv
    d1 = outv - yv
    d2 = xv - qv
    s1 = jnp.sum(d1 * d1)
    s2 = jnp.sum(d2 * d2)

    @pl.when(i == 0)
    def _():
        acc_ref[0] = s1
        acc_ref[1] = s2

    @pl.when(i > 0)
    def _():
        acc_ref[0] = acc_ref[0] + s1
        acc_ref[1] = acc_ref[1] + s2

    @pl.when(i == _N // _R - 1)
    def _():
        n = float(_N * _C)
        loss_ref[0, 0] = acc_ref[0] / n + (1.0 + _BETA) * (acc_ref[1] / n)


def _out_and_loss(flat_x, flat_y, q):
    return pl.pallas_call(
        _loss_body,
        grid=(_N // _R,),
        in_specs=[
            pl.BlockSpec((_R, _C), lambda i: (i, 0)),
            pl.BlockSpec((_R, _C), lambda i: (i, 0)),
            pl.BlockSpec((_R, _C), lambda i: (i, 0)),
        ],
        out_specs=[
            pl.BlockSpec((_R, _C), lambda i: (i, 0)),
            pl.BlockSpec(memory_space=pltpu.SMEM),
        ],
        out_shape=[
            jax.ShapeDtypeStruct((_N, _C), jnp.float32),
            jax.ShapeDtypeStruct((1, 1), jnp.float32),
        ],
        scratch_shapes=[pltpu.SMEM((2,), jnp.float32)],
        compiler_params=pltpu.CompilerParams(
            dimension_semantics=("arbitrary",)
        ),
    )(flat_x, flat_y, q)


def kernel(x, y, out_codebook, quantization_noise_std):
    flat = x.reshape(_N, _C)
    inds = _nearest_inds(flat, out_codebook)
    q = _gather_sc(out_codebook, inds.reshape(_N))
    out_flat, loss = _out_and_loss(flat, y.reshape(_N, _C), q)
    return out_flat.reshape(_B, _E, _C), loss[0, 0]
